# Initial kernel scaffold; baseline (speedup 1.0000x reference)
import functools

import numpy as np
import jax
import jax.numpy as jnp
from jax import lax
from jax.experimental import pallas as pl
from jax.experimental.pallas import tpu as pltpu
from jax.experimental.pallas import tpu_sc as plsc

_SAMPLING_P = 0.1
_LANES = 16


@functools.lru_cache(maxsize=None)
def _plan(B, T):
    k = int(0.8 * T)
    base = jax.random.key(42)
    coin_key, perm_key = jax.random.split(base)
    coins = np.asarray(jax.random.uniform(coin_key, (B,)))
    perm_keys = jax.random.split(perm_key, B)
    idx = np.asarray(
        jax.vmap(lambda pk: jnp.sort(jax.random.permutation(pk, T)[:k]))(perm_keys)
    )
    hit = coins < _SAMPLING_P
    src = np.tile(np.arange(T, dtype=np.int64), (B, 1))
    src[hit, :k] = idx[hit]
    src[hit, k:] = 0
    g = (src + (np.arange(B, dtype=np.int64) * T)[:, None]).astype(np.int32)
    ts = np.where(hit, 0.0, 1.0).astype(np.float32)
    return g.reshape(-1), ts


@functools.lru_cache(maxsize=None)
def _build(B, T, C, NC, NS):
    NW = NC * NS
    R = B * T
    k = int(0.8 * T)
    GN = 128
    CH = 512
    CPW = R // NW
    BPW = B // NW
    n_ch = CPW // CH
    n_g = CH // GN
    assert CPW * NW == R and BPW * NW == B and T % CH == 0 and C % _LANES == 0

    mesh = plsc.VectorSubcoreMesh(core_axis_name="c", subcore_axis_name="s")

    @functools.partial(
        pl.kernel,
        out_type=jax.ShapeDtypeStruct((R, C), jnp.float32),
        mesh=mesh,
        scratch_types=[
            pltpu.VMEM((CPW // GN, GN), jnp.int32),
            pltpu.VMEM((CH, C), jnp.float32),
            pltpu.VMEM((B,), jnp.float32),
            pltpu.SemaphoreType.DMA,
        ],
    )
    def sc_kernel(x_hbm, g_hbm, ts_hbm, out_hbm, idx_v, rows_v, ts_v, sem):
        wid = lax.axis_index("s") * NC + lax.axis_index("c")
        pltpu.sync_copy(g_hbm.at[pl.ds(wid * (CPW // GN), CPW // GN)], idx_v)
        pltpu.sync_copy(ts_hbm, ts_v)
        base = wid * CPW
        for s in range(n_ch):
            cps = [
                pltpu.async_copy(
                    x_hbm.at[idx_v.at[s * n_g + j]],
                    rows_v.at[pl.ds(j * GN, GN)],
                    sem,
                )
                for j in range(n_g)
            ]
            for cp in cps:
                cp.wait()
            for i in range(BPW):
                lo = max(s * CH, i * T + k)
                hi = min((s + 1) * CH, (i + 1) * T)
                if lo < hi:
                    bidx = wid * BPW + i
                    s_vec = plsc.load_gather(
                        ts_v, [jnp.full((_LANES,), bidx, jnp.int32)]
                    )

                    def body(r, carry, s_vec=s_vec):
                        row = rows_v.at[r]
                        for cix in range(C // _LANES):
                            sl = pl.ds(cix * _LANES, _LANES)
                            row[sl] = row[sl] * s_vec
                        return carry

                    lax.fori_loop(lo - s * CH, hi - s * CH, body, 0)
            pltpu.sync_copy(rows_v, out_hbm.at[pl.ds(base + s * CH, CH)])

    return sc_kernel


def kernel(eeg_data):
    B, T, C = eeg_data.shape
    g, ts = _plan(B, T)
    info = plsc.get_sparse_core_info()
    sck = _build(B, T, C, info.num_cores, info.num_subcores)
    x_flat = eeg_data.reshape(B * T, C)
    out = sck(x_flat, jnp.asarray(g.reshape(-1, 128)), jnp.asarray(ts))
    return out.reshape(B, T, C)


# speedup vs baseline: 1.4782x; 1.4782x over previous
import functools

import numpy as np
import jax
import jax.numpy as jnp
from jax import lax
from jax.experimental import pallas as pl
from jax.experimental.pallas import tpu as pltpu
from jax.experimental.pallas import tpu_sc as plsc

_SAMPLING_P = 0.1
_LANES = 16


def _rotl32(x, r):
    return ((x << np.uint32(r)) | (x >> np.uint32(32 - r))).astype(np.uint32)


def _tf2x32(k1, k2, x0, x1):
    x0 = x0.astype(np.uint32).copy()
    x1 = x1.astype(np.uint32).copy()
    ks = (np.uint32(k1), np.uint32(k2),
          np.uint32(k1) ^ np.uint32(k2) ^ np.uint32(0x1BD11BDA))
    rot_a = (13, 15, 26, 6)
    rot_b = (17, 29, 16, 24)
    x0 = (x0 + ks[0]).astype(np.uint32)
    x1 = (x1 + ks[1]).astype(np.uint32)
    sched = [(rot_a, ks[1], ks[2] + np.uint32(1)),
             (rot_b, ks[2], ks[0] + np.uint32(2)),
             (rot_a, ks[0], ks[1] + np.uint32(3)),
             (rot_b, ks[1], ks[2] + np.uint32(4)),
             (rot_a, ks[2], ks[0] + np.uint32(5))]
    for rots, a0, a1 in sched:
        for r in rots:
            x0 = (x0 + x1).astype(np.uint32)
            x1 = _rotl32(x1, r)
            x1 = x1 ^ x0
        x0 = (x0 + a0).astype(np.uint32)
        x1 = (x1 + a1).astype(np.uint32)
    return x0, x1


def _np_tf_2x32(key, count):
    flat = count.astype(np.uint32).ravel()
    odd = flat.size % 2
    if odd:
        flat = np.concatenate([flat, np.zeros(1, np.uint32)])
    h = flat.size // 2
    r0, r1 = _tf2x32(key[0], key[1], flat[:h], flat[h:])
    out = np.concatenate([r0, r1])
    if odd:
        out = out[:-1]
    return out.reshape(count.shape)


def _np_partitionable():
    return bool(jax.config.jax_threefry_partitionable)


def _np_split(key, n=2):
    if _np_partitionable():
        b1, b2 = _tf2x32(key[0], key[1], np.zeros(n, np.uint32),
                         np.arange(n, dtype=np.uint32))
        return np.stack([b1, b2], axis=1)
    return _np_tf_2x32(key, np.arange(2 * n, dtype=np.uint32)).reshape(n, 2)


def _np_bits32(key, n):
    if _np_partitionable():
        b1, b2 = _tf2x32(key[0], key[1], np.zeros(n, np.uint32),
                         np.arange(n, dtype=np.uint32))
        return b1 ^ b2
    return _np_tf_2x32(key, np.arange(n, dtype=np.uint32))


def _np_uniform(key, n):
    bits = _np_bits32(key, n)
    f = ((bits >> np.uint32(9)) | np.uint32(0x3F800000)).view(np.float32)
    return np.maximum(np.float32(0.0), f - np.float32(1.0))


def _np_permutation(key, T):
    x = np.arange(T, dtype=np.int32)
    num_rounds = int(np.ceil(3 * np.log(max(1, T)) /
                             np.log(np.iinfo(np.uint32).max)))
    k = key
    for _ in range(num_rounds):
        ks = _np_split(k, 2)
        k, sub = ks[0], ks[1]
        sort_keys = _np_bits32(sub, T)
        x = x[np.argsort(sort_keys, kind="stable")]
    return x


@functools.lru_cache(maxsize=None)
def _plan(B, T):
    k = int(0.8 * T)
    base = np.array([0, 42], dtype=np.uint32)
    keys = _np_split(base, 2)
    coin_key, perm_key = keys[0], keys[1]
    coins = _np_uniform(coin_key, B)
    perm_keys = _np_split(perm_key, B)
    idx = np.stack([np.sort(_np_permutation(perm_keys[i], T)[:k])
                    for i in range(B)])
    hit = coins < _SAMPLING_P
    src = np.tile(np.arange(T, dtype=np.int64), (B, 1))
    src[hit, :k] = idx[hit]
    src[hit, k:] = 0
    g = (src + (np.arange(B, dtype=np.int64) * T)[:, None]).astype(np.int32)
    ts = np.repeat(np.where(hit, 0.0, 1.0).astype(np.float32)[:, None], _LANES, axis=1)
    return g.reshape(-1), ts


@functools.lru_cache(maxsize=None)
def _build(B, T, C, NC, NS):
    NW = NC * NS
    R = B * T
    k = int(0.8 * T)
    GN = 128
    CH = 512
    CPW = R // NW
    BPW = B // NW
    n_ch = CPW // CH
    n_g = CH // GN
    assert CPW * NW == R and BPW * NW == B and T % CH == 0 and C % _LANES == 0

    mesh = plsc.VectorSubcoreMesh(core_axis_name="c", subcore_axis_name="s")

    @functools.partial(
        pl.kernel,
        out_type=jax.ShapeDtypeStruct((R, C), jnp.float32),
        mesh=mesh,
        compiler_params=pltpu.CompilerParams(use_tc_tiling_on_sc=False),
        scratch_types=[
            pltpu.VMEM((CPW // GN, GN), jnp.int32),
            pltpu.VMEM((CH, C), jnp.float32),
            pltpu.VMEM((B, _LANES), jnp.float32),
            pltpu.SemaphoreType.DMA,
        ],
    )
    def sc_kernel(x_hbm, g_hbm, ts_hbm, out_hbm, idx_v, rows_v, ts_v, sem):
        wid = lax.axis_index("s") * NC + lax.axis_index("c")
        pltpu.sync_copy(g_hbm.at[pl.ds(wid * (CPW // GN), CPW // GN)], idx_v)
        pltpu.sync_copy(ts_hbm, ts_v)
        base = wid * CPW
        for s in range(n_ch):
            cps = [
                pltpu.async_copy(
                    x_hbm.at[idx_v.at[s * n_g + j]],
                    rows_v.at[pl.ds(j * GN, GN)],
                    sem,
                )
                for j in range(n_g)
            ]
            for cp in cps:
                cp.wait()
            for i in range(BPW):
                lo = max(s * CH, i * T + k)
                hi = min((s + 1) * CH, (i + 1) * T)
                if lo < hi:
                    bidx = wid * BPW + i
                    s_vec = ts_v[bidx, :]

                    def body(r, carry, s_vec=s_vec):
                        row = rows_v.at[r]
                        for cix in range(C // _LANES):
                            sl = pl.ds(cix * _LANES, _LANES)
                            row[sl] = row[sl] * s_vec
                        return carry

                    lax.fori_loop(lo - s * CH, hi - s * CH, body, 0)
            pltpu.sync_copy(rows_v, out_hbm.at[pl.ds(base + s * CH, CH)])

    return sc_kernel


def kernel(eeg_data):
    B, T, C = eeg_data.shape
    g, ts = _plan(B, T)
    info = plsc.get_sparse_core_info()
    sck = _build(B, T, C, info.num_cores, info.num_subcores)
    x_flat = eeg_data.reshape(B * T, C)
    out = sck(x_flat, jnp.asarray(g.reshape(-1, 128)), jnp.asarray(ts))
    return out.reshape(B, T, C)


# speedup vs baseline: 1.4814x; 1.0021x over previous
import functools

import numpy as np
import jax
import jax.numpy as jnp
from jax import lax
from jax.experimental import pallas as pl
from jax.experimental.pallas import tpu as pltpu
from jax.experimental.pallas import tpu_sc as plsc

_SAMPLING_P = 0.1
_LANES = 16


def _rotl32(x, r):
    return ((x << np.uint32(r)) | (x >> np.uint32(32 - r))).astype(np.uint32)


def _tf2x32(k1, k2, x0, x1):
    x0 = x0.astype(np.uint32).copy()
    x1 = x1.astype(np.uint32).copy()
    ks = (np.uint32(k1), np.uint32(k2),
          np.uint32(k1) ^ np.uint32(k2) ^ np.uint32(0x1BD11BDA))
    rot_a = (13, 15, 26, 6)
    rot_b = (17, 29, 16, 24)
    x0 = (x0 + ks[0]).astype(np.uint32)
    x1 = (x1 + ks[1]).astype(np.uint32)
    sched = [(rot_a, ks[1], ks[2] + np.uint32(1)),
             (rot_b, ks[2], ks[0] + np.uint32(2)),
             (rot_a, ks[0], ks[1] + np.uint32(3)),
             (rot_b, ks[1], ks[2] + np.uint32(4)),
             (rot_a, ks[2], ks[0] + np.uint32(5))]
    for rots, a0, a1 in sched:
        for r in rots:
            x0 = (x0 + x1).astype(np.uint32)
            x1 = _rotl32(x1, r)
            x1 = x1 ^ x0
        x0 = (x0 + a0).astype(np.uint32)
        x1 = (x1 + a1).astype(np.uint32)
    return x0, x1


def _np_tf_2x32(key, count):
    flat = count.astype(np.uint32).ravel()
    odd = flat.size % 2
    if odd:
        flat = np.concatenate([flat, np.zeros(1, np.uint32)])
    h = flat.size // 2
    r0, r1 = _tf2x32(key[0], key[1], flat[:h], flat[h:])
    out = np.concatenate([r0, r1])
    if odd:
        out = out[:-1]
    return out.reshape(count.shape)


def _np_partitionable():
    return bool(jax.config.jax_threefry_partitionable)


def _np_split(key, n=2):
    if _np_partitionable():
        b1, b2 = _tf2x32(key[0], key[1], np.zeros(n, np.uint32),
                         np.arange(n, dtype=np.uint32))
        return np.stack([b1, b2], axis=1)
    return _np_tf_2x32(key, np.arange(2 * n, dtype=np.uint32)).reshape(n, 2)


def _np_bits32(key, n):
    if _np_partitionable():
        b1, b2 = _tf2x32(key[0], key[1], np.zeros(n, np.uint32),
                         np.arange(n, dtype=np.uint32))
        return b1 ^ b2
    return _np_tf_2x32(key, np.arange(n, dtype=np.uint32))


def _np_uniform(key, n):
    bits = _np_bits32(key, n)
    f = ((bits >> np.uint32(9)) | np.uint32(0x3F800000)).view(np.float32)
    return np.maximum(np.float32(0.0), f - np.float32(1.0))


def _np_permutation(key, T):
    x = np.arange(T, dtype=np.int32)
    num_rounds = int(np.ceil(3 * np.log(max(1, T)) /
                             np.log(np.iinfo(np.uint32).max)))
    k = key
    for _ in range(num_rounds):
        ks = _np_split(k, 2)
        k, sub = ks[0], ks[1]
        sort_keys = _np_bits32(sub, T)
        x = x[np.argsort(sort_keys, kind="stable")]
    return x


@functools.lru_cache(maxsize=None)
def _plan(B, T):
    k = int(0.8 * T)
    base = np.array([0, 42], dtype=np.uint32)
    keys = _np_split(base, 2)
    coin_key, perm_key = keys[0], keys[1]
    coins = _np_uniform(coin_key, B)
    perm_keys = _np_split(perm_key, B)
    idx = np.stack([np.sort(_np_permutation(perm_keys[i], T)[:k])
                    for i in range(B)])
    hit = coins < _SAMPLING_P
    src = np.tile(np.arange(T, dtype=np.int64), (B, 1))
    src[hit, :k] = idx[hit]
    src[hit, k:] = 0
    g = src.astype(np.int32)
    ts = np.repeat(np.where(hit, 0.0, 1.0).astype(np.float32)[:, None], _LANES, axis=1)
    return g, ts


@functools.lru_cache(maxsize=None)
def _build(B, T, C, NC, NS):
    NW = NC * NS
    k = int(0.8 * T)
    GN = 128
    CH = 512
    BPW = B // NW
    n_ch = T // CH
    n_g = CH // GN
    assert BPW * NW == B and T % CH == 0 and C % _LANES == 0

    mesh = plsc.VectorSubcoreMesh(core_axis_name="c", subcore_axis_name="s")

    @functools.partial(
        pl.kernel,
        out_type=jax.ShapeDtypeStruct((B, T, C), jnp.float32),
        mesh=mesh,
        compiler_params=pltpu.CompilerParams(use_tc_tiling_on_sc=False),
        scratch_types=[
            pltpu.VMEM((T // GN, GN), jnp.int32),
            pltpu.VMEM((CH, C), jnp.float32),
            pltpu.VMEM((B, _LANES), jnp.float32),
            pltpu.SemaphoreType.DMA,
        ],
    )
    def sc_kernel(x_hbm, g_hbm, ts_hbm, out_hbm, idx_v, rows_v, ts_v, sem):
        wid = lax.axis_index("s") * NC + lax.axis_index("c")
        pltpu.sync_copy(ts_hbm, ts_v)
        for i in range(BPW):
            b = wid * BPW + i
            pltpu.sync_copy(g_hbm.at[b], idx_v)
            s_vec = ts_v[b, :]
            xb = x_hbm.at[b]
            for s in range(n_ch):
                cps = [
                    pltpu.async_copy(
                        xb.at[idx_v.at[s * n_g + j]],
                        rows_v.at[pl.ds(j * GN, GN)],
                        sem,
                    )
                    for j in range(n_g)
                ]
                for cp in cps:
                    cp.wait()
                lo = max(s * CH, k)
                hi = (s + 1) * CH
                if lo < hi:

                    def body(r, carry, s_vec=s_vec):
                        row = rows_v.at[r]
                        for cix in range(C // _LANES):
                            sl = pl.ds(cix * _LANES, _LANES)
                            row[sl] = row[sl] * s_vec
                        return carry

                    lax.fori_loop(lo - s * CH, hi - s * CH, body, 0)
                pltpu.sync_copy(rows_v, out_hbm.at[b, pl.ds(s * CH, CH), :])

    return sc_kernel


def kernel(eeg_data):
    B, T, C = eeg_data.shape
    g, ts = _plan(B, T)
    info = plsc.get_sparse_core_info()
    sck = _build(B, T, C, info.num_cores, info.num_subcores)
    out = sck(eeg_data,
              jnp.asarray(g.reshape(B, T // 128, 128)),
              jnp.asarray(ts))
    return out
